# baseline (speedup 1.0000x reference)
import functools

import jax
import jax.numpy as jnp
from jax import lax
from jax.experimental import pallas as pl
from jax.experimental.pallas import tpu as pltpu
from jax.experimental.pallas import tpu_sc as plsc

B = 1024
N = 8192
K = 256
NV = N // 16
NC, NS, L = 2, 16, 16
NW = NC * NS
ROWS_PER_W = B // NW
CAP = N + 16

_MESH = plsc.VectorSubcoreMesh(
    core_axis_name="c", subcore_axis_name="s", num_cores=NC, num_subcores=NS)


def _suffix_scan(h0, h1):
    c1 = plsc.cumsum(lax.rev(h1, (0,)))
    s1 = lax.rev(c1, (0,))
    t1 = jnp.max(c1)
    c0 = plsc.cumsum(lax.rev(h0, (0,)))
    s0 = lax.rev(c0, (0,)) + t1
    return s0, s1


@functools.partial(
    pl.kernel,
    out_type=jax.ShapeDtypeStruct((B, 2 * K), jnp.float32),
    mesh=_MESH,
    compiler_params=pltpu.CompilerParams(needs_layout_passes=False),
    scratch_types=[
        pltpu.VMEM((2 * 2 * N,), jnp.float32),
        pltpu.VMEM((N,), jnp.int32),
        pltpu.VMEM((128,), jnp.int32),
        pltpu.VMEM((NV,), jnp.int32),
        pltpu.VMEM((NV,), jnp.int32),
        pltpu.VMEM((CAP,), jnp.int32),
        pltpu.VMEM((CAP,), jnp.int32),
        pltpu.VMEM((CAP,), jnp.int32),
        pltpu.VMEM((CAP,), jnp.int32),
        pltpu.VMEM((2 * K,), jnp.float32),
        pltpu.SemaphoreType.DMA,
        pltpu.SemaphoreType.DMA,
    ],
)
def _topk_sc(dgm_hbm, out_hbm, dgm2, key_v, hist, cnts, bases,
             ck0, ci0, ck1, ci1, outv, sem0, sem1):
    wid = lax.axis_index("s") * NC + lax.axis_index("c")
    row0 = wid * ROWS_PER_W
    iota = lax.iota(jnp.int32, L)
    zeros16 = jnp.zeros((L,), jnp.int32)
    ones16 = jnp.ones((L,), jnp.int32)
    lane0 = iota == 0
    sems = (sem0, sem1)
    dbufs = (dgm2.at[pl.ds(0, 2 * N)], dgm2.at[pl.ds(2 * N, 2 * N)])

    pltpu.make_async_copy(
        dgm_hbm.at[pl.ds(row0 * 2 * N, 2 * N)], dbufs[0], sems[0]).start()

    def do_row(r, par, dv, sem):
        row = row0 + r
        pltpu.make_async_copy(
            dgm_hbm.at[pl.ds(row * 2 * N, 2 * N)], dv, sem).wait()

        @pl.when(r < ROWS_PER_W - 1)
        def _():
            pltpu.make_async_copy(
                dgm_hbm.at[pl.ds((row + 1) * 2 * N, 2 * N)],
                dbufs[1 - par], sems[1 - par]).start()

        for j in range(8):
            hist[pl.ds(16 * j, 16)] = zeros16

        @plsc.parallel_loop(0, NV, unroll=4)
        def _p1(i):
            base = i * L
            rows16 = (base + iota) * 2
            bb = plsc.load_gather(dv, [rows16])
            dd = plsc.load_gather(dv, [rows16 + 1])
            p = dd - bb
            kb = plsc.bitcast(p, jnp.int32)
            key = kb ^ ((kb >> 31) | jnp.int32(-2**31))
            key_v[pl.ds(base, 16)] = key
            dig = (key >> 27) & 31
            plsc.addupdate_scatter(hist, [dig + ((i & 3) << 5)], ones16)

        h0 = (hist[pl.ds(0, 16)] + hist[pl.ds(32, 16)]
              + hist[pl.ds(64, 16)] + hist[pl.ds(96, 16)])
        h1 = (hist[pl.ds(16, 16)] + hist[pl.ds(48, 16)]
              + hist[pl.ds(80, 16)] + hist[pl.ds(112, 16)])
        s0, s1 = _suffix_scan(h0, h1)
        d0 = jnp.max(jnp.where(s0 >= K, iota, -1))
        d1 = jnp.max(jnp.where(s1 >= K, iota + 16, -1))
        dstar = jnp.maximum(d0, d1)

        @plsc.parallel_loop(0, NV, unroll=4)
        def _p2a(c):
            key = key_v[pl.ds(c * L, 16)]
            dig = (key >> 27) & 31
            si = jnp.where(dig >= dstar, 1, 0)
            tot0 = lax.rev(plsc.cumsum(si), (0,))
            plsc.store_scatter(cnts, [jnp.full((L,), c, jnp.int32)],
                               tot0, mask=lane0)

        def p2b(i, run):
            c = cnts[pl.ds(i * L, 16)]
            pc = plsc.cumsum(c)
            bases[pl.ds(i * L, 16)] = run + pc - c
            return run + jnp.max(pc)

        m = lax.fori_loop(0, NV // L, p2b, jnp.int32(0))

        @plsc.parallel_loop(0, NV, unroll=4)
        def _p2c(c):
            key = key_v[pl.ds(c * L, 16)]
            dig = (key >> 27) & 31
            msk = dig >= dstar
            pref = plsc.cumsum(jnp.where(msk, 1, 0))
            bsp = plsc.load_gather(bases, [jnp.full((L,), c, jnp.int32)])
            pos = jnp.where(msk, bsp + pref - 1, CAP - 1)
            plsc.store_scatter(ck0, [pos], key)
            plsc.store_scatter(ci0, [pos], c * L + iota)

        ck0[pl.ds(m, 16)] = zeros16
        ci0[pl.ds(m, 16)] = zeros16
        trips = (m + 15) >> 4

        bufs = ((ck0, ci0), (ck1, ci1))
        for p in range(7):
            sk, si_ = bufs[p % 2]
            dk, di_ = bufs[(p + 1) % 2]
            sh = 5 * p

            hist[pl.ds(0, 16)] = zeros16
            hist[pl.ds(16, 16)] = zeros16

            @plsc.parallel_loop(0, trips)
            def _hcount(i, sk=sk, sh=sh):
                key = sk[pl.ds(i * L, 16)]
                dig = (key >> sh) & 31
                plsc.addupdate_scatter(hist, [dig], ones16)

            h0 = hist[pl.ds(0, 16)]
            h1 = hist[pl.ds(16, 16)]
            s0, s1 = _suffix_scan(h0, h1)
            hist[pl.ds(0, 16)] = s0 - h0
            hist[pl.ds(16, 16)] = s1 - h1

            def perm(i, _c, sk=sk, si_=si_, dk=dk, di_=di_, sh=sh):
                key = sk[pl.ds(i * L, 16)]
                idxv = si_[pl.ds(i * L, 16)]
                dig = (key >> sh) & 31
                cnt, last = plsc.scan_count(dig)
                basev = plsc.load_gather(hist, [dig])
                pos = basev + cnt - 1
                plsc.store_scatter(dk, [pos], key)
                plsc.store_scatter(di_, [pos], idxv)
                plsc.addupdate_scatter(hist, [dig], cnt, mask=last)
                return 0

            lax.fori_loop(0, trips, perm, 0)

        @plsc.parallel_loop(0, K // L, unroll=4)
        def _emit(t):
            pos16 = t * L + iota
            sidx = ci1[pl.ds(t * L, 16)] * 2
            bb = plsc.load_gather(dv, [sidx])
            dd = plsc.load_gather(dv, [sidx + 1])
            plsc.store_scatter(outv, [2 * pos16], bb)
            plsc.store_scatter(outv, [2 * pos16 + 1], dd)
        pltpu.sync_copy(outv, out_hbm.at[row])

    def do2(rr, _c):
        for par in range(2):
            do_row(2 * rr + par, par, dbufs[par], sems[par])
        return 0

    lax.fori_loop(0, ROWS_PER_W // 2, do2, 0)


def kernel(diagrams):
    return _topk_sc(diagrams.reshape(B * 2 * N))


# speedup vs baseline: 19.5684x; 19.5684x over previous
import functools

import jax
import jax.numpy as jnp
from jax import lax
from jax.experimental import pallas as pl
from jax.experimental.pallas import tpu as pltpu
from jax.experimental.pallas import tpu_sc as plsc

B = 1024
N = 8192
K = 256
NV = N // 16
NC, NS, L = 2, 16, 16
NW = NC * NS
ROWS_PER_W = B // NW
CAP = N + 16

_MESH = plsc.VectorSubcoreMesh(
    core_axis_name="c", subcore_axis_name="s", num_cores=NC, num_subcores=NS)


def _suffix_scan(h0, h1):
    c1 = plsc.cumsum(lax.rev(h1, (0,)))
    s1 = lax.rev(c1, (0,))
    t1 = jnp.max(c1)
    c0 = plsc.cumsum(lax.rev(h0, (0,)))
    s0 = lax.rev(c0, (0,)) + t1
    return s0, s1


@functools.partial(
    pl.kernel,
    out_type=jax.ShapeDtypeStruct((B, 2 * K), jnp.float32),
    mesh=_MESH,
    compiler_params=pltpu.CompilerParams(needs_layout_passes=False),
    scratch_types=[
        pltpu.VMEM((2 * 2 * N,), jnp.float32),
        pltpu.VMEM((N,), jnp.int32),
        pltpu.VMEM((4096,), jnp.int32),
        pltpu.VMEM((NV,), jnp.int32),
        pltpu.VMEM((NV,), jnp.int32),
        pltpu.VMEM((CAP,), jnp.int32),
        pltpu.VMEM((CAP,), jnp.int32),
        pltpu.VMEM((CAP,), jnp.int32),
        pltpu.VMEM((CAP,), jnp.int32),
        pltpu.VMEM((2 * K,), jnp.float32),
        pltpu.SemaphoreType.DMA,
        pltpu.SemaphoreType.DMA,
    ],
)
def _topk_sc(dgm_hbm, out_hbm, dgm2, key_v, hist, cnts, bases,
             ck0, ci0, ck1, ci1, outv, sem0, sem1):
    wid = lax.axis_index("s") * NC + lax.axis_index("c")
    row0 = wid * ROWS_PER_W
    iota = lax.iota(jnp.int32, L)
    zeros16 = jnp.zeros((L,), jnp.int32)
    ones16 = jnp.ones((L,), jnp.int32)
    lane0 = iota == 0
    sems = (sem0, sem1)
    dbufs = (dgm2.at[pl.ds(0, 2 * N)], dgm2.at[pl.ds(2 * N, 2 * N)])

    pltpu.make_async_copy(dgm_hbm.at[row0], dbufs[0], sems[0]).start()

    def do_row(r, par, dv, sem):
        row = row0 + r
        pltpu.make_async_copy(dgm_hbm.at[row], dv, sem).wait()

        @pl.when(r < ROWS_PER_W - 1)
        def _():
            pltpu.make_async_copy(
                dgm_hbm.at[row + 1], dbufs[1 - par], sems[1 - par]).start()

        @plsc.parallel_loop(0, 256, unroll=4)
        def _hzero(i):
            hist[pl.ds(i * L, 16)] = zeros16

        @plsc.parallel_loop(0, NV, unroll=4)
        def _p1(i):
            base = i * L
            rows16 = (base + iota) * 2
            bb = plsc.load_gather(dv, [rows16])
            dd = plsc.load_gather(dv, [rows16 + 1])
            p = dd - bb
            kb = plsc.bitcast(p, jnp.int32)
            key = kb ^ ((kb >> 31) | jnp.int32(-2**31))
            key_v[pl.ds(base, 16)] = key
            dig = (key >> 22) & 1023
            plsc.addupdate_scatter(hist, [dig + ((i & 3) << 10)], ones16)

        @plsc.parallel_loop(0, 64, unroll=4)
        def _gmerge(g):
            hs = (hist[pl.ds(g * L, 16)] + hist[pl.ds(1024 + g * L, 16)]
                  + hist[pl.ds(2048 + g * L, 16)] + hist[pl.ds(3072 + g * L, 16)])
            hist[pl.ds(g * L, 16)] = hs
            tot0 = lax.rev(plsc.cumsum(hs), (0,))
            plsc.store_scatter(cnts, [jnp.full((L,), g, jnp.int32)],
                               tot0, mask=lane0)

        carry = jnp.int32(0)
        sg, gv = [None] * 4, [None] * 4
        for cc in (3, 2, 1, 0):
            v = cnts[pl.ds(cc * L, 16)]
            s = lax.rev(plsc.cumsum(lax.rev(v, (0,))), (0,)) + carry
            gv[cc], sg[cc] = v, s
            carry = jnp.max(s)
        cstar = jnp.int32(-1)
        sgval = jnp.int32(0)
        gtotc = jnp.int32(0)
        for cc in range(4):
            cstar = jnp.maximum(
                cstar, jnp.max(jnp.where(sg[cc] >= K, iota + 16 * cc, -1)))
        for cc in range(4):
            sel = (iota + 16 * cc) == cstar
            sgval = sgval + jnp.max(jnp.where(sel, sg[cc], 0))
            gtotc = gtotc + jnp.max(jnp.where(sel, gv[cc], 0))
        above = sgval - gtotc
        h16 = hist[pl.ds(cstar * L, 16)]
        sin = lax.rev(plsc.cumsum(lax.rev(h16, (0,))), (0,)) + above
        dstar = cstar * L + jnp.max(jnp.where(sin >= K, iota, -1))

        @plsc.parallel_loop(0, NV, unroll=4)
        def _p2a(c):
            key = key_v[pl.ds(c * L, 16)]
            dig = (key >> 22) & 1023
            si = jnp.where(dig >= dstar, 1, 0)
            tot0 = lax.rev(plsc.cumsum(si), (0,))
            plsc.store_scatter(cnts, [jnp.full((L,), c, jnp.int32)],
                               tot0, mask=lane0)

        def p2b(i, run):
            c = cnts[pl.ds(i * L, 16)]
            pc = plsc.cumsum(c)
            bases[pl.ds(i * L, 16)] = run + pc - c
            return run + jnp.max(pc)

        m = lax.fori_loop(0, NV // L, p2b, jnp.int32(0))

        @plsc.parallel_loop(0, NV, unroll=4)
        def _p2c(c):
            key = key_v[pl.ds(c * L, 16)]
            dig = (key >> 22) & 1023
            msk = dig >= dstar
            pref = plsc.cumsum(jnp.where(msk, 1, 0))
            bsp = plsc.load_gather(bases, [jnp.full((L,), c, jnp.int32)])
            pos = jnp.where(msk, bsp + pref - 1, CAP - 1)
            plsc.store_scatter(ck0, [pos], key)
            plsc.store_scatter(ci0, [pos], c * L + iota)

        ck0[pl.ds(m, 16)] = zeros16
        ci0[pl.ds(m, 16)] = zeros16
        trips = (m + 15) >> 4

        bufs = ((ck0, ci0), (ck1, ci1))
        for p in range(7):
            sk, si_ = bufs[p % 2]
            dk, di_ = bufs[(p + 1) % 2]
            sh = 5 * p

            hist[pl.ds(0, 16)] = zeros16
            hist[pl.ds(16, 16)] = zeros16

            @plsc.parallel_loop(0, trips)
            def _hcount(i, sk=sk, sh=sh):
                key = sk[pl.ds(i * L, 16)]
                dig = (key >> sh) & 31
                plsc.addupdate_scatter(hist, [dig], ones16)

            h0 = hist[pl.ds(0, 16)]
            h1 = hist[pl.ds(16, 16)]
            s0, s1 = _suffix_scan(h0, h1)
            hist[pl.ds(0, 16)] = s0 - h0
            hist[pl.ds(16, 16)] = s1 - h1

            def perm(i, _c, sk=sk, si_=si_, dk=dk, di_=di_, sh=sh):
                key = sk[pl.ds(i * L, 16)]
                idxv = si_[pl.ds(i * L, 16)]
                dig = (key >> sh) & 31
                cnt, last = plsc.scan_count(dig)
                basev = plsc.load_gather(hist, [dig])
                pos = basev + cnt - 1
                plsc.store_scatter(dk, [pos], key)
                plsc.store_scatter(di_, [pos], idxv)
                plsc.addupdate_scatter(hist, [dig], cnt, mask=last)
                return 0

            lax.fori_loop(0, trips, perm, 0)

        @plsc.parallel_loop(0, K // L, unroll=4)
        def _emit(t):
            pos16 = t * L + iota
            sidx = ci1[pl.ds(t * L, 16)] * 2
            bb = plsc.load_gather(dv, [sidx])
            dd = plsc.load_gather(dv, [sidx + 1])
            plsc.store_scatter(outv, [2 * pos16], bb)
            plsc.store_scatter(outv, [2 * pos16 + 1], dd)
        pltpu.sync_copy(outv, out_hbm.at[row])

    def do2(rr, _c):
        for par in range(2):
            do_row(2 * rr + par, par, dbufs[par], sems[par])
        return 0

    lax.fori_loop(0, ROWS_PER_W // 2, do2, 0)


def kernel(diagrams):
    return _topk_sc(diagrams.reshape(B, 2 * N))


# speedup vs baseline: 19.8871x; 1.0163x over previous
import functools

import jax
import jax.numpy as jnp
from jax import lax
from jax.experimental import pallas as pl
from jax.experimental.pallas import tpu as pltpu
from jax.experimental.pallas import tpu_sc as plsc

B = 1024
N = 8192
K = 256
NV = N // 16
NC, NS, L = 2, 16, 16
NW = NC * NS
ROWS_PER_W = B // NW
CAP = N + 16

_MESH = plsc.VectorSubcoreMesh(
    core_axis_name="c", subcore_axis_name="s", num_cores=NC, num_subcores=NS)


def _suffix_scan(h0, h1):
    c1 = plsc.cumsum(lax.rev(h1, (0,)))
    s1 = lax.rev(c1, (0,))
    t1 = jnp.max(c1)
    c0 = plsc.cumsum(lax.rev(h0, (0,)))
    s0 = lax.rev(c0, (0,)) + t1
    return s0, s1


@functools.partial(
    pl.kernel,
    out_type=jax.ShapeDtypeStruct((B, 2 * K), jnp.float32),
    mesh=_MESH,
    compiler_params=pltpu.CompilerParams(needs_layout_passes=False),
    scratch_types=[
        pltpu.VMEM((2 * 2 * N,), jnp.float32),
        pltpu.VMEM((N,), jnp.int32),
        pltpu.VMEM((4096,), jnp.int32),
        pltpu.VMEM((NV,), jnp.int32),
        pltpu.VMEM((NV,), jnp.int32),
        pltpu.VMEM((CAP,), jnp.int32),
        pltpu.VMEM((CAP,), jnp.int32),
        pltpu.VMEM((CAP,), jnp.int32),
        pltpu.VMEM((CAP,), jnp.int32),
        pltpu.VMEM((2 * K,), jnp.float32),
        pltpu.VMEM((2 * K,), jnp.float32),
        pltpu.SemaphoreType.DMA,
        pltpu.SemaphoreType.DMA,
        pltpu.SemaphoreType.DMA,
        pltpu.SemaphoreType.DMA,
    ],
)
def _topk_sc(dgm_hbm, out_hbm, dgm2, key_v, hist, cnts, bases,
             ck0, ci0, ck1, ci1, outva, outvb, sem0, sem1, semo0, semo1):
    wid = lax.axis_index("s") * NC + lax.axis_index("c")
    row0 = wid * ROWS_PER_W
    iota = lax.iota(jnp.int32, L)
    zeros16 = jnp.zeros((L,), jnp.int32)
    ones16 = jnp.ones((L,), jnp.int32)
    lane0 = iota == 0
    sems = (sem0, sem1)
    semos = (semo0, semo1)
    dbufs = (dgm2.at[pl.ds(0, 2 * N)], dgm2.at[pl.ds(2 * N, 2 * N)])
    obufs = (outva, outvb)

    pltpu.make_async_copy(dgm_hbm.at[row0], dbufs[0], sems[0]).start()

    def do_row(r, par, dv, sem, ov, semo):
        row = row0 + r
        pltpu.make_async_copy(dgm_hbm.at[row], dv, sem).wait()

        @pl.when(r < ROWS_PER_W - 1)
        def _():
            pltpu.make_async_copy(
                dgm_hbm.at[row + 1], dbufs[1 - par], sems[1 - par]).start()

        @plsc.parallel_loop(0, 256, unroll=4)
        def _hzero(i):
            hist[pl.ds(i * L, 16)] = zeros16

        @plsc.parallel_loop(0, NV, unroll=8)
        def _p1(i):
            base = i * L
            rows16 = (base + iota) * 2
            bb = plsc.load_gather(dv, [rows16])
            dd = plsc.load_gather(dv, [rows16 + 1])
            p = dd - bb
            kb = plsc.bitcast(p, jnp.int32)
            key = kb ^ ((kb >> 31) | jnp.int32(-2**31))
            key_v[pl.ds(base, 16)] = key
            dig = (key >> 22) & 1023
            plsc.addupdate_scatter(hist, [dig + ((i & 3) << 10)], ones16)

        @plsc.parallel_loop(0, 64, unroll=4)
        def _gmerge(g):
            hs = (hist[pl.ds(g * L, 16)] + hist[pl.ds(1024 + g * L, 16)]
                  + hist[pl.ds(2048 + g * L, 16)] + hist[pl.ds(3072 + g * L, 16)])
            hist[pl.ds(g * L, 16)] = hs
            tot0 = lax.rev(plsc.cumsum(hs), (0,))
            plsc.store_scatter(cnts, [jnp.full((L,), g, jnp.int32)],
                               tot0, mask=lane0)

        carry = jnp.int32(0)
        sg, gv = [None] * 4, [None] * 4
        for cc in (3, 2, 1, 0):
            v = cnts[pl.ds(cc * L, 16)]
            s = lax.rev(plsc.cumsum(lax.rev(v, (0,))), (0,)) + carry
            gv[cc], sg[cc] = v, s
            carry = jnp.max(s)
        cstar = jnp.int32(-1)
        sgval = jnp.int32(0)
        gtotc = jnp.int32(0)
        for cc in range(4):
            cstar = jnp.maximum(
                cstar, jnp.max(jnp.where(sg[cc] >= K, iota + 16 * cc, -1)))
        for cc in range(4):
            sel = (iota + 16 * cc) == cstar
            sgval = sgval + jnp.max(jnp.where(sel, sg[cc], 0))
            gtotc = gtotc + jnp.max(jnp.where(sel, gv[cc], 0))
        above = sgval - gtotc
        h16 = hist[pl.ds(cstar * L, 16)]
        sin = lax.rev(plsc.cumsum(lax.rev(h16, (0,))), (0,)) + above
        dstar = cstar * L + jnp.max(jnp.where(sin >= K, iota, -1))

        @plsc.parallel_loop(0, NV, unroll=8)
        def _p2a(c):
            key = key_v[pl.ds(c * L, 16)]
            dig = (key >> 22) & 1023
            si = jnp.where(dig >= dstar, 1, 0)
            tot0 = lax.rev(plsc.cumsum(si), (0,))
            plsc.store_scatter(cnts, [jnp.full((L,), c, jnp.int32)],
                               tot0, mask=lane0)

        def p2b(i, run):
            c = cnts[pl.ds(i * L, 16)]
            pc = plsc.cumsum(c)
            bases[pl.ds(i * L, 16)] = run + pc - c
            return run + jnp.max(pc)

        m = lax.fori_loop(0, NV // L, p2b, jnp.int32(0))

        @plsc.parallel_loop(0, NV, unroll=8)
        def _p2c(c):
            key = key_v[pl.ds(c * L, 16)]
            dig = (key >> 22) & 1023
            msk = dig >= dstar
            pref = plsc.cumsum(jnp.where(msk, 1, 0))
            bsp = plsc.load_gather(bases, [jnp.full((L,), c, jnp.int32)])
            pos = jnp.where(msk, bsp + pref - 1, CAP - 1)
            plsc.store_scatter(ck0, [pos], key)
            plsc.store_scatter(ci0, [pos], c * L + iota)

        ck0[pl.ds(m, 16)] = zeros16
        ci0[pl.ds(m, 16)] = zeros16
        trips = (m + 15) >> 4

        bufs = ((ck0, ci0), (ck1, ci1))
        for p in range(7):
            sk, si_ = bufs[p % 2]
            dk, di_ = bufs[(p + 1) % 2]
            sh = 5 * p

            hist[pl.ds(0, 16)] = zeros16
            hist[pl.ds(16, 16)] = zeros16

            @plsc.parallel_loop(0, trips)
            def _hcount(i, sk=sk, sh=sh):
                key = sk[pl.ds(i * L, 16)]
                dig = (key >> sh) & 31
                plsc.addupdate_scatter(hist, [dig], ones16)

            h0 = hist[pl.ds(0, 16)]
            h1 = hist[pl.ds(16, 16)]
            s0, s1 = _suffix_scan(h0, h1)
            hist[pl.ds(0, 16)] = s0 - h0
            hist[pl.ds(16, 16)] = s1 - h1

            def perm(i, _c, sk=sk, si_=si_, dk=dk, di_=di_, sh=sh):
                key = sk[pl.ds(i * L, 16)]
                idxv = si_[pl.ds(i * L, 16)]
                dig = (key >> sh) & 31
                cnt, last = plsc.scan_count(dig)
                basev = plsc.load_gather(hist, [dig])
                pos = basev + cnt - 1
                plsc.store_scatter(dk, [pos], key)
                plsc.store_scatter(di_, [pos], idxv)
                plsc.addupdate_scatter(hist, [dig], cnt, mask=last)
                return 0

            lax.fori_loop(0, trips, perm, 0)

        @pl.when(r >= 2)
        def _():
            pltpu.make_async_copy(ov, out_hbm.at[row - 2], semo).wait()

        @plsc.parallel_loop(0, K // L, unroll=4)
        def _emit(t):
            pos16 = t * L + iota
            sidx = ci1[pl.ds(t * L, 16)] * 2
            bb = plsc.load_gather(dv, [sidx])
            dd = plsc.load_gather(dv, [sidx + 1])
            plsc.store_scatter(ov, [2 * pos16], bb)
            plsc.store_scatter(ov, [2 * pos16 + 1], dd)

        pltpu.make_async_copy(ov, out_hbm.at[row], semo).start()

    def do2(rr, _c):
        for par in range(2):
            do_row(2 * rr + par, par, dbufs[par], sems[par],
                   obufs[par], semos[par])
        return 0

    lax.fori_loop(0, ROWS_PER_W // 2, do2, 0)

    for par in range(2):
        pltpu.make_async_copy(
            obufs[par], out_hbm.at[row0 + ROWS_PER_W - 2 + par],
            semos[par]).wait()


def kernel(diagrams):
    return _topk_sc(diagrams.reshape(B, 2 * N))
